# baseline (speedup 1.0000x reference)
import functools

import jax
import jax.numpy as jnp
from jax import lax
from jax.experimental import pallas as pl
from jax.experimental.pallas import tpu as pltpu
from jax.experimental.pallas import tpu_sc as plsc

D = 128
N_WORKERS = 32
N_SUB = 16
CHUNK = 128
NBUF = 4
LA = 2
VIA = (False, True, False, True)


def _emb_kernel(n_total):
    per_w = n_total // N_WORKERS
    n_chunks = per_w // CHUNK
    mesh = plsc.VectorSubcoreMesh(core_axis_name="c", subcore_axis_name="s")

    @functools.partial(
        pl.kernel,
        mesh=mesh,
        out_type=jax.ShapeDtypeStruct((n_total, D), jnp.float32),
        scratch_types=[
            pltpu.VMEM((n_chunks, CHUNK), jnp.int32),
            pltpu.VMEM((NBUF, CHUNK, D), jnp.float32),
            pltpu.VMEM_SHARED((N_SUB, 2, CHUNK, D), jnp.float32),
            pltpu.SemaphoreType.DMA,
            pltpu.SemaphoreType.DMA,
            pltpu.SemaphoreType.DMA,
            pltpu.SemaphoreType.DMA,
            pltpu.SemaphoreType.DMA,
            pltpu.SemaphoreType.DMA,
            pltpu.SemaphoreType.DMA,
            pltpu.SemaphoreType.DMA,
        ],
    )
    def k(idx_hbm, tbl_hbm, out_hbm, idx_v, rows_v, sp,
          g0, g1, g2, g3, s0, s1, h1, h3):
        gsem = (g0, g1, g2, g3)
        ssem = {0: s0, 2: s1}
        hsem = {1: h1, 3: h3}
        cid = lax.axis_index("c")
        sid = lax.axis_index("s")
        wid = sid * 2 + cid
        base = wid * per_w

        pltpu.sync_copy(idx_hbm.at[wid], idx_v)

        def start_gather(c, b):
            pltpu.async_copy(tbl_hbm.at[idx_v.at[c]], rows_v.at[b], gsem[b])

        for c in range(LA):
            start_gather(c, c % NBUF)

        def body(g, carry):
            for b in range(NBUF):
                c = g * NBUF + b
                pltpu.make_async_copy(
                    tbl_hbm.at[idx_v.at[c]], rows_v.at[b], gsem[b]
                ).wait()
                out_slc = out_hbm.at[pl.ds(base + c * CHUNK, CHUNK)]
                if VIA[b]:
                    j = b // 2
                    spb = sp.at[sid, j]
                    @pl.when(c >= NBUF)
                    def _():
                        prev = out_hbm.at[
                            pl.ds(base + (c - NBUF) * CHUNK, CHUNK)
                        ]
                        pltpu.make_async_copy(spb, prev, hsem[b]).wait()
                    pltpu.sync_copy(rows_v.at[b], spb)
                    pltpu.async_copy(spb, out_slc, hsem[b])
                else:
                    pltpu.async_copy(rows_v.at[b], out_slc, ssem[b])

                nb = (b + LA) % NBUF

                @pl.when(c + LA < n_chunks)
                def _():
                    if not VIA[nb]:
                        pc = c + LA - NBUF
                        @pl.when(pc >= 0)
                        def _():
                            prev = out_hbm.at[pl.ds(base + pc * CHUNK, CHUNK)]
                            pltpu.make_async_copy(
                                rows_v.at[nb], prev, ssem[nb]
                            ).wait()
                    start_gather(c + LA, nb)

            return carry

        lax.fori_loop(0, n_chunks // NBUF, body, 0)

        for c in range(n_chunks - NBUF, n_chunks):
            b = c % NBUF
            out_slc = out_hbm.at[pl.ds(base + c * CHUNK, CHUNK)]
            if VIA[b]:
                pltpu.make_async_copy(sp.at[sid, b // 2], out_slc,
                                      hsem[b]).wait()
            else:
                pltpu.make_async_copy(rows_v.at[b], out_slc, ssem[b]).wait()

    return k


def kernel(x, weight):
    b, s = x.shape
    n_total = b * s
    per_w = n_total // N_WORKERS
    idx = x.reshape(N_WORKERS, per_w // CHUNK, CHUNK).astype(jnp.int32)
    out = _emb_kernel(n_total)(idx, weight)
    return out.reshape(b, s, weight.shape[1])


# speedup vs baseline: 1.0129x; 1.0129x over previous
import functools

import jax
import jax.numpy as jnp
from jax import lax
from jax.experimental import pallas as pl
from jax.experimental.pallas import tpu as pltpu
from jax.experimental.pallas import tpu_sc as plsc

D = 128
N_WORKERS = 32
CHUNK = 128
NBUF = 5
LA = 3


def _emb_kernel(n_total):
    per_w = n_total // N_WORKERS
    n_chunks = per_w // CHUNK
    mesh = plsc.VectorSubcoreMesh(core_axis_name="c", subcore_axis_name="s")

    @functools.partial(
        pl.kernel,
        mesh=mesh,
        out_type=jax.ShapeDtypeStruct((n_total, D), jnp.float32),
        scratch_types=[
            pltpu.VMEM((n_chunks, CHUNK), jnp.int32),
            pltpu.VMEM((NBUF, CHUNK, D), jnp.float32),
            pltpu.SemaphoreType.DMA,
            pltpu.SemaphoreType.DMA,
            pltpu.SemaphoreType.DMA,
            pltpu.SemaphoreType.DMA,
            pltpu.SemaphoreType.DMA,
            pltpu.SemaphoreType.DMA,
            pltpu.SemaphoreType.DMA,
            pltpu.SemaphoreType.DMA,
            pltpu.SemaphoreType.DMA,
            pltpu.SemaphoreType.DMA,
        ],
    )
    def k(idx_hbm, tbl_hbm, out_hbm, idx_v, rows_v,
          g0, g1, g2, g3, g4, s0, s1, s2, s3, s4):
        gsem = (g0, g1, g2, g3, g4)
        ssem = (s0, s1, s2, s3, s4)
        wid = lax.axis_index("s") * 2 + lax.axis_index("c")
        base = wid * per_w

        pltpu.sync_copy(idx_hbm.at[wid], idx_v)

        def start_gather(c, b):
            pltpu.async_copy(tbl_hbm.at[idx_v.at[c]], rows_v.at[b], gsem[b])

        for c in range(LA):
            start_gather(c, c % NBUF)

        def body(g, carry):
            for b in range(NBUF):
                c = g * NBUF + b
                pltpu.make_async_copy(
                    tbl_hbm.at[idx_v.at[c]], rows_v.at[b], gsem[b]
                ).wait()
                out_slc = out_hbm.at[pl.ds(base + c * CHUNK, CHUNK)]
                pltpu.async_copy(rows_v.at[b], out_slc, ssem[b])

                nb = (b + LA) % NBUF

                @pl.when(c + LA < n_chunks)
                def _():
                    pc = c + LA - NBUF
                    @pl.when(pc >= 0)
                    def _():
                        prev = out_hbm.at[pl.ds(base + pc * CHUNK, CHUNK)]
                        pltpu.make_async_copy(
                            rows_v.at[nb], prev, ssem[nb]
                        ).wait()
                    start_gather(c + LA, nb)

            return carry

        lax.fori_loop(0, n_chunks // NBUF, body, 0)

        for c in range(n_chunks - NBUF, n_chunks):
            b = c % NBUF
            out_slc = out_hbm.at[pl.ds(base + c * CHUNK, CHUNK)]
            pltpu.make_async_copy(rows_v.at[b], out_slc, ssem[b]).wait()

    return k


def kernel(x, weight):
    b, s = x.shape
    n_total = b * s
    per_w = n_total // N_WORKERS
    idx = x.reshape(N_WORKERS, per_w // CHUNK, CHUNK).astype(jnp.int32)
    out = _emb_kernel(n_total)(idx, weight)
    return out.reshape(b, s, weight.shape[1])
